# baseline (speedup 1.0000x reference)
import functools

import jax
import jax.numpy as jnp
from jax import lax
from jax.experimental import pallas as pl
from jax.experimental.pallas import tpu as pltpu, tpu_sc as plsc

B = 16384
D_CHAMP = 32
D_MISC = 16
H1 = 256
H2 = 128
N_CSLOT = 10
N_MSLOT = 5
NG = 4

NC = 2
NS = 16
NW = NC * NS
BPW = B // NW

BM = 512


def _sc_gather(ctab, cidx, mtab, midx):
    mesh = plsc.VectorSubcoreMesh(core_axis_name="c", subcore_axis_name="s")

    @functools.partial(
        pl.kernel,
        mesh=mesh,
        compiler_params=pltpu.CompilerParams(use_tc_tiling_on_sc=False),
        out_type=jax.ShapeDtypeStruct((NG, B, 128), jnp.float32),
        scratch_types=[
            pltpu.VMEM((BPW,), jnp.int32),
            pltpu.VMEM((BPW, D_CHAMP), jnp.float32),
            pltpu.VMEM((BPW, D_MISC), jnp.float32),
            pltpu.SemaphoreType.DMA,
        ],
    )
    def k(ctab_hbm, cidx_hbm, mtab_hbm, midx_hbm, xp_hbm,
          idx_v, crows_v, mrows_v, sem):
        wid = lax.axis_index("s") * NC + lax.axis_index("c")
        base = wid * BPW
        for s in range(N_CSLOT):
            pltpu.sync_copy(cidx_hbm.at[pl.ds(s * B + base, BPW)], idx_v)
            pltpu.async_copy(ctab_hbm.at[idx_v], crows_v, sem).wait()
            pltpu.sync_copy(
                crows_v,
                xp_hbm.at[s // 4, pl.ds(base, BPW),
                          pl.ds((s % 4) * D_CHAMP, D_CHAMP)])
        for m in range(N_MSLOT):
            pltpu.sync_copy(midx_hbm.at[pl.ds(m * B + base, BPW)], idx_v)
            pltpu.async_copy(mtab_hbm.at[idx_v], mrows_v, sem).wait()
            pltpu.sync_copy(
                mrows_v,
                xp_hbm.at[3, pl.ds(base, BPW), pl.ds(m * D_MISC, D_MISC)])

    return k(ctab, cidx, mtab, midx)


def _mlp_body(xp_ref, w1c_ref, w1m_ref, b1_ref, w2_ref, b2_ref,
              w3_ref, b3_ref, out_ref):
    h = jnp.broadcast_to(b1_ref[...], (BM, H1))
    for r in range(3):
        band = xp_ref[r]
        for q in range(4):
            s = 4 * r + q
            if s >= N_CSLOT:
                break
            piece = band[:, q * D_CHAMP : (q + 1) * D_CHAMP]
            h = h + jnp.dot(piece, w1c_ref[s])
    mband = xp_ref[3]
    for m in range(N_MSLOT):
        piece = mband[:, m * D_MISC : (m + 1) * D_MISC]
        h = h + jnp.dot(piece, w1m_ref[m])
    h = jnp.maximum(h, 0.0)
    h2 = jnp.maximum(jnp.dot(h, w2_ref[...]) + b2_ref[...], 0.0)
    out_ref[...] = jnp.dot(h2, w3_ref[...]) + b3_ref[...]


def _tc_mlp(xp, W1, b1, W2, b2, W3, b3):
    W1c = W1[: N_CSLOT * D_CHAMP].reshape(N_CSLOT, D_CHAMP, H1)
    W1m = W1[N_CSLOT * D_CHAMP :].reshape(N_MSLOT, D_MISC, H1)
    out = pl.pallas_call(
        _mlp_body,
        grid=(B // BM,),
        in_specs=[
            pl.BlockSpec((NG, BM, 128), lambda i: (0, i, 0)),
            pl.BlockSpec((N_CSLOT, D_CHAMP, H1), lambda i: (0, 0, 0)),
            pl.BlockSpec((N_MSLOT, D_MISC, H1), lambda i: (0, 0, 0)),
            pl.BlockSpec((1, H1), lambda i: (0, 0)),
            pl.BlockSpec((H1, H2), lambda i: (0, 0)),
            pl.BlockSpec((1, H2), lambda i: (0, 0)),
            pl.BlockSpec((H2, 1), lambda i: (0, 0)),
            pl.BlockSpec((1, 1), lambda i: (0, 0)),
        ],
        out_specs=pl.BlockSpec((BM, 1), lambda i: (i, 0)),
        out_shape=jax.ShapeDtypeStruct((B, 1), jnp.float32),
    )(xp, W1c, W1m, b1.reshape(1, H1), W2, b2.reshape(1, H2), W3,
      b3.reshape(1, 1))
    return out[:, 0]


def kernel(my_idx, ally_lists, enem_lists, misc_idx, emb_champ, emb_sp,
           emb_pri, emb_sub, emb_key, emb_pat, W1, b1, W2, b2, W3, b3):
    cidx = jnp.concatenate(
        [my_idx[None, :], ally_lists, enem_lists], axis=0
    ).astype(jnp.int32).reshape(N_CSLOT * B)
    mtab = jnp.concatenate([emb_sp, emb_pri, emb_sub, emb_key, emb_pat], axis=0)
    midx = (
        misc_idx.astype(jnp.int32)
        + jnp.arange(N_MSLOT, dtype=jnp.int32)[None, :] * emb_sp.shape[0]
    ).T.reshape(N_MSLOT * B)
    xp = _sc_gather(emb_champ, cidx, mtab, midx)
    return _tc_mlp(xp, W1, b1, W2, b2, W3, b3)


# speedup vs baseline: 1.2349x; 1.2349x over previous
import functools

import jax
import jax.numpy as jnp
from jax import lax
from jax.experimental import pallas as pl
from jax.experimental.pallas import tpu as pltpu, tpu_sc as plsc

B = 16384
D_CHAMP = 32
D_MISC = 16
H1 = 256
H2 = 128
N_CSLOT = 10
N_MSLOT = 5
NG = 4

NC = 2
NS = 16
NW = NC * NS
BPW = B // NW
RING = 3

BM = 512


def _sc_gather(ctab, cidxw, mtab, midxw):
    mesh = plsc.VectorSubcoreMesh(core_axis_name="c", subcore_axis_name="s")

    @functools.partial(
        pl.kernel,
        mesh=mesh,
        compiler_params=pltpu.CompilerParams(use_tc_tiling_on_sc=False),
        out_type=jax.ShapeDtypeStruct((NG, B, 128), jnp.float32),
        scratch_types=[
            pltpu.VMEM((N_CSLOT * BPW,), jnp.int32),
            pltpu.VMEM((N_MSLOT * BPW,), jnp.int32),
            pltpu.VMEM((RING, BPW, D_CHAMP), jnp.float32),
            pltpu.VMEM((RING, BPW, D_MISC), jnp.float32),
            pltpu.SemaphoreType.DMA,
            pltpu.SemaphoreType.DMA,
        ],
    )
    def k(ctab_hbm, cidxw_hbm, mtab_hbm, midxw_hbm, xp_hbm,
          idxc_v, idxm_v, crows_v, mrows_v, sg, sw):
        wid = lax.axis_index("s") * NC + lax.axis_index("c")
        base = wid * BPW
        pltpu.sync_copy(
            cidxw_hbm.at[pl.ds(wid * N_CSLOT * BPW, N_CSLOT * BPW)], idxc_v)
        pltpu.sync_copy(
            midxw_hbm.at[pl.ds(wid * N_MSLOT * BPW, N_MSLOT * BPW)], idxm_v)

        def cgather(s):
            return pltpu.async_copy(
                ctab_hbm.at[idxc_v.at[pl.ds(s * BPW, BPW)]],
                crows_v.at[s % RING], sg)

        def cwrite(s, lane, width=D_CHAMP, buf=None):
            return pltpu.async_copy(
                crows_v.at[buf if buf is not None else s % RING],
                xp_hbm.at[s // 4, pl.ds(base, BPW), pl.ds(lane, width)], sw)

        def mgather(m):
            return pltpu.async_copy(
                mtab_hbm.at[idxm_v.at[pl.ds(m * BPW, BPW)]],
                mrows_v.at[m % RING], sg)

        def mwrite(m, lane, buf=None):
            return pltpu.async_copy(
                mrows_v.at[buf if buf is not None else m % RING],
                xp_hbm.at[3, pl.ds(base, BPW), pl.ds(lane, D_MISC)], sw)

        pend = []
        gd = {s: cgather(s) for s in range(RING)}
        for s in range(N_CSLOT):
            gd[s].wait()
            w = cwrite(s, (s % 4) * D_CHAMP)
            if s + RING < N_CSLOT:
                w.wait()
                gd[s + RING] = cgather(s + RING)
            else:
                pend.append(w)
        gm = {m: mgather(m) for m in range(RING)}
        pend.append(cwrite(8, 64, buf=8 % RING))
        pend.append(cwrite(8, 96, buf=8 % RING))
        for m in range(N_MSLOT):
            gm[m].wait()
            w = mwrite(m, m * D_MISC)
            if m + RING < N_MSLOT:
                w.wait()
                gm[m + RING] = mgather(m + RING)
            else:
                pend.append(w)
        pend.append(mwrite(4, 80, buf=4 % RING))
        pend.append(mwrite(4, 96, buf=4 % RING))
        pend.append(mwrite(4, 112, buf=4 % RING))
        for w in pend:
            w.wait()

    return k(ctab, cidxw, mtab, midxw)


def _mlp_body(xp_ref, w1x_ref, b1_ref, w2_ref, b2_ref, w3_ref, b3_ref,
              out_ref):
    x = jnp.concatenate(
        [xp_ref[0], xp_ref[1], xp_ref[2], xp_ref[3]], axis=-1)
    h = jnp.maximum(jnp.dot(x, w1x_ref[...]) + b1_ref[...], 0.0)
    h2 = jnp.maximum(jnp.dot(h, w2_ref[...]) + b2_ref[...], 0.0)
    out_ref[...] = jnp.dot(h2, w3_ref[...]) + b3_ref[...]


def _tc_mlp(xp, W1, b1, W2, b2, W3, b3):
    z64 = jnp.zeros((64, H1), jnp.float32)
    z48 = jnp.zeros((48, H1), jnp.float32)
    W1x = jnp.concatenate([W1[:320], z64, W1[320:], z48], axis=0)
    out = pl.pallas_call(
        _mlp_body,
        grid=(B // BM,),
        in_specs=[
            pl.BlockSpec((NG, BM, 128), lambda i: (0, i, 0)),
            pl.BlockSpec((NG * 128, H1), lambda i: (0, 0)),
            pl.BlockSpec((1, H1), lambda i: (0, 0)),
            pl.BlockSpec((H1, H2), lambda i: (0, 0)),
            pl.BlockSpec((1, H2), lambda i: (0, 0)),
            pl.BlockSpec((H2, 1), lambda i: (0, 0)),
            pl.BlockSpec((1, 1), lambda i: (0, 0)),
        ],
        out_specs=pl.BlockSpec((BM, 1), lambda i: (i, 0)),
        out_shape=jax.ShapeDtypeStruct((B, 1), jnp.float32),
    )(xp, W1x, b1.reshape(1, H1), W2, b2.reshape(1, H2), W3,
      b3.reshape(1, 1))
    return out[:, 0]


def kernel(my_idx, ally_lists, enem_lists, misc_idx, emb_champ, emb_sp,
           emb_pri, emb_sub, emb_key, emb_pat, W1, b1, W2, b2, W3, b3):
    cidxw = (
        jnp.concatenate([my_idx[None, :], ally_lists, enem_lists], axis=0)
        .astype(jnp.int32)
        .reshape(N_CSLOT, NW, BPW)
        .swapaxes(0, 1)
        .reshape(NW * N_CSLOT * BPW)
    )
    mtab = jnp.concatenate([emb_sp, emb_pri, emb_sub, emb_key, emb_pat], axis=0)
    midxw = (
        (misc_idx.astype(jnp.int32)
         + jnp.arange(N_MSLOT, dtype=jnp.int32)[None, :] * emb_sp.shape[0])
        .T.reshape(N_MSLOT, NW, BPW)
        .swapaxes(0, 1)
        .reshape(NW * N_MSLOT * BPW)
    )
    xp = _sc_gather(emb_champ, cidxw, mtab, midxw)
    return _tc_mlp(xp, W1, b1, W2, b2, W3, b3)


# speedup vs baseline: 1.2529x; 1.0146x over previous
import functools

import jax
import jax.numpy as jnp
from jax import lax
from jax.experimental import pallas as pl
from jax.experimental.pallas import tpu as pltpu, tpu_sc as plsc

B = 16384
D_CHAMP = 32
D_MISC = 16
H1 = 256
H2 = 128
N_CSLOT = 10
N_MSLOT = 5
NG = 4

NC = 2
NS = 16
NW = NC * NS
NCHUNK = 2
BCH = B // NCHUNK
BPW = BCH // NW
RING = 3

BM = 512


def _sc_gather(ctab, cidxw, mtab, midxw):
    mesh = plsc.VectorSubcoreMesh(core_axis_name="c", subcore_axis_name="s")

    @functools.partial(
        pl.kernel,
        mesh=mesh,
        compiler_params=pltpu.CompilerParams(use_tc_tiling_on_sc=False),
        out_type=jax.ShapeDtypeStruct((NG, BCH, 128), jnp.float32),
        scratch_types=[
            pltpu.VMEM((N_CSLOT * BPW,), jnp.int32),
            pltpu.VMEM((N_MSLOT * BPW,), jnp.int32),
            pltpu.VMEM((RING, BPW, D_CHAMP), jnp.float32),
            pltpu.VMEM((RING, BPW, D_MISC), jnp.float32),
            pltpu.SemaphoreType.DMA,
            pltpu.SemaphoreType.DMA,
        ],
    )
    def k(ctab_hbm, cidxw_hbm, mtab_hbm, midxw_hbm, xp_hbm,
          idxc_v, idxm_v, crows_v, mrows_v, sg, sw):
        wid = lax.axis_index("s") * NC + lax.axis_index("c")
        base = wid * BPW
        pltpu.sync_copy(
            cidxw_hbm.at[pl.ds(wid * N_CSLOT * BPW, N_CSLOT * BPW)], idxc_v)
        pltpu.sync_copy(
            midxw_hbm.at[pl.ds(wid * N_MSLOT * BPW, N_MSLOT * BPW)], idxm_v)

        def cgather(s):
            return pltpu.async_copy(
                ctab_hbm.at[idxc_v.at[pl.ds(s * BPW, BPW)]],
                crows_v.at[s % RING], sg)

        def cwrite(s, lane, width=D_CHAMP, buf=None):
            return pltpu.async_copy(
                crows_v.at[buf if buf is not None else s % RING],
                xp_hbm.at[s // 4, pl.ds(base, BPW), pl.ds(lane, width)], sw)

        def mgather(m):
            return pltpu.async_copy(
                mtab_hbm.at[idxm_v.at[pl.ds(m * BPW, BPW)]],
                mrows_v.at[m % RING], sg)

        def mwrite(m, lane, buf=None):
            return pltpu.async_copy(
                mrows_v.at[buf if buf is not None else m % RING],
                xp_hbm.at[3, pl.ds(base, BPW), pl.ds(lane, D_MISC)], sw)

        pend = []
        gd = {s: cgather(s) for s in range(RING)}
        for s in range(N_CSLOT):
            gd[s].wait()
            w = cwrite(s, (s % 4) * D_CHAMP)
            if s + RING < N_CSLOT:
                w.wait()
                gd[s + RING] = cgather(s + RING)
            else:
                pend.append(w)
        gm = {m: mgather(m) for m in range(RING)}
        pend.append(cwrite(8, 64, buf=8 % RING))
        pend.append(cwrite(8, 96, buf=8 % RING))
        for m in range(N_MSLOT):
            gm[m].wait()
            w = mwrite(m, m * D_MISC)
            if m + RING < N_MSLOT:
                w.wait()
                gm[m + RING] = mgather(m + RING)
            else:
                pend.append(w)
        pend.append(mwrite(4, 80, buf=4 % RING))
        pend.append(mwrite(4, 96, buf=4 % RING))
        pend.append(mwrite(4, 112, buf=4 % RING))
        for w in pend:
            w.wait()

    return k(ctab, cidxw, mtab, midxw)


def _mlp_body(xp_ref, w1x_ref, b1_ref, w2_ref, b2_ref, w3_ref, b3_ref,
              out_ref):
    x = jnp.concatenate(
        [xp_ref[0], xp_ref[1], xp_ref[2], xp_ref[3]], axis=-1)
    h = jnp.maximum(jnp.dot(x, w1x_ref[...]) + b1_ref[...], 0.0)
    h2 = jnp.maximum(jnp.dot(h, w2_ref[...]) + b2_ref[...], 0.0)
    out_ref[...] = jnp.dot(h2, w3_ref[...]) + b3_ref[...]


def _tc_mlp(xp, W1, b1, W2, b2, W3, b3):
    z64 = jnp.zeros((64, H1), jnp.float32)
    z48 = jnp.zeros((48, H1), jnp.float32)
    W1x = jnp.concatenate([W1[:320], z64, W1[320:], z48], axis=0)
    out = pl.pallas_call(
        _mlp_body,
        grid=(BCH // BM,),
        in_specs=[
            pl.BlockSpec((NG, BM, 128), lambda i: (0, i, 0)),
            pl.BlockSpec((NG * 128, H1), lambda i: (0, 0)),
            pl.BlockSpec((1, H1), lambda i: (0, 0)),
            pl.BlockSpec((H1, H2), lambda i: (0, 0)),
            pl.BlockSpec((1, H2), lambda i: (0, 0)),
            pl.BlockSpec((H2, 1), lambda i: (0, 0)),
            pl.BlockSpec((1, 1), lambda i: (0, 0)),
        ],
        out_specs=pl.BlockSpec((BM, 1), lambda i: (i, 0)),
        out_shape=jax.ShapeDtypeStruct((BCH, 1), jnp.float32),
    )(xp, W1x, b1.reshape(1, H1), W2, b2.reshape(1, H2), W3,
      b3.reshape(1, 1))
    return out[:, 0]


def kernel(my_idx, ally_lists, enem_lists, misc_idx, emb_champ, emb_sp,
           emb_pri, emb_sub, emb_key, emb_pat, W1, b1, W2, b2, W3, b3):
    cidxw = (
        jnp.concatenate([my_idx[None, :], ally_lists, enem_lists], axis=0)
        .astype(jnp.int32)
        .reshape(N_CSLOT, NCHUNK, NW, BPW)
        .transpose(1, 2, 0, 3)
        .reshape(NCHUNK, NW * N_CSLOT * BPW)
    )
    mtab = jnp.concatenate([emb_sp, emb_pri, emb_sub, emb_key, emb_pat], axis=0)
    midxw = (
        (misc_idx.astype(jnp.int32)
         + jnp.arange(N_MSLOT, dtype=jnp.int32)[None, :] * emb_sp.shape[0])
        .T.reshape(N_MSLOT, NCHUNK, NW, BPW)
        .transpose(1, 2, 0, 3)
        .reshape(NCHUNK, NW * N_MSLOT * BPW)
    )
    xps = [
        _sc_gather(emb_champ, cidxw[c], mtab, midxw[c])
        for c in range(NCHUNK)
    ]
    outs = [_tc_mlp(xp, W1, b1, W2, b2, W3, b3) for xp in xps]
    return jnp.concatenate(outs, axis=0)


# speedup vs baseline: 1.2544x; 1.0012x over previous
import functools

import jax
import jax.numpy as jnp
from jax import lax
from jax.experimental import pallas as pl
from jax.experimental.pallas import tpu as pltpu, tpu_sc as plsc

B = 16384
D_CHAMP = 32
D_MISC = 16
H1 = 256
H2 = 128
N_CSLOT = 10
N_MSLOT = 5
NG = 4

NC = 2
NS = 16
NW = NC * NS
NCHUNK = 2
BCH = B // NCHUNK
BPW = BCH // NW
RING = 3

BM = 512


def _sc_gather(ctab, cidx, mtab, midx, chunk):
    mesh = plsc.VectorSubcoreMesh(core_axis_name="c", subcore_axis_name="s")

    @functools.partial(
        pl.kernel,
        mesh=mesh,
        compiler_params=pltpu.CompilerParams(use_tc_tiling_on_sc=False),
        out_type=jax.ShapeDtypeStruct((NG, BCH, 128), jnp.float32),
        scratch_types=[
            pltpu.VMEM((N_CSLOT, BPW), jnp.int32),
            pltpu.VMEM((N_MSLOT, BPW), jnp.int32),
            pltpu.VMEM((RING, BPW, D_CHAMP), jnp.float32),
            pltpu.VMEM((RING, BPW, D_MISC), jnp.float32),
            pltpu.SemaphoreType.DMA,
            pltpu.SemaphoreType.DMA,
            pltpu.SemaphoreType.DMA,
        ],
    )
    def k(ctab_hbm, cidx_hbm, mtab_hbm, midx_hbm, xp_hbm,
          idxc_v, idxm_v, crows_v, mrows_v, sg, sw, si):
        wid = lax.axis_index("s") * NC + lax.axis_index("c")
        base = wid * BPW
        idescs = []
        for s in range(N_CSLOT):
            idescs.append(pltpu.async_copy(
                cidx_hbm.at[pl.ds(s * B + chunk * BCH + base, BPW)],
                idxc_v.at[s], si))
        for m in range(N_MSLOT):
            idescs.append(pltpu.async_copy(
                midx_hbm.at[pl.ds(m * B + chunk * BCH + base, BPW)],
                idxm_v.at[m], si))

        def cgather(s):
            return pltpu.async_copy(
                ctab_hbm.at[idxc_v.at[s]],
                crows_v.at[s % RING], sg)

        def cwrite(s, lane, width=D_CHAMP, buf=None):
            return pltpu.async_copy(
                crows_v.at[buf if buf is not None else s % RING],
                xp_hbm.at[s // 4, pl.ds(base, BPW), pl.ds(lane, width)], sw)

        def mgather(m):
            return pltpu.async_copy(
                mtab_hbm.at[idxm_v.at[m]],
                mrows_v.at[m % RING], sg)

        def mwrite(m, lane, buf=None):
            return pltpu.async_copy(
                mrows_v.at[buf if buf is not None else m % RING],
                xp_hbm.at[3, pl.ds(base, BPW), pl.ds(lane, D_MISC)], sw)

        pend = []
        gd = {}
        for s in range(RING):
            idescs[s].wait()
            gd[s] = cgather(s)
        for s in range(RING, N_CSLOT):
            idescs[s].wait()
        for m in range(N_MSLOT):
            idescs[N_CSLOT + m].wait()
        for s in range(N_CSLOT):
            gd[s].wait()
            w = cwrite(s, (s % 4) * D_CHAMP)
            if s + RING < N_CSLOT:
                w.wait()
                gd[s + RING] = cgather(s + RING)
            else:
                pend.append(w)
        gm = {m: mgather(m) for m in range(RING)}
        pend.append(cwrite(8, 64, buf=8 % RING))
        pend.append(cwrite(8, 96, buf=8 % RING))
        for m in range(N_MSLOT):
            gm[m].wait()
            w = mwrite(m, m * D_MISC)
            if m + RING < N_MSLOT:
                w.wait()
                gm[m + RING] = mgather(m + RING)
            else:
                pend.append(w)
        pend.append(mwrite(4, 80, buf=4 % RING))
        pend.append(mwrite(4, 96, buf=4 % RING))
        pend.append(mwrite(4, 112, buf=4 % RING))
        for w in pend:
            w.wait()

    return k(ctab, cidx, mtab, midx)


def _mlp_body(xp_ref, w1x_ref, b1_ref, w2_ref, b2_ref, w3_ref, b3_ref,
              out_ref):
    x = jnp.concatenate(
        [xp_ref[0], xp_ref[1], xp_ref[2], xp_ref[3]], axis=-1)
    h = jnp.maximum(jnp.dot(x, w1x_ref[...]) + b1_ref[...], 0.0)
    h2 = jnp.maximum(jnp.dot(h, w2_ref[...]) + b2_ref[...], 0.0)
    out_ref[...] = jnp.dot(h2, w3_ref[...]) + b3_ref[...]


def _tc_mlp(xp, W1, b1, W2, b2, W3, b3):
    z64 = jnp.zeros((64, H1), jnp.float32)
    z48 = jnp.zeros((48, H1), jnp.float32)
    W1x = jnp.concatenate([W1[:320], z64, W1[320:], z48], axis=0)
    out = pl.pallas_call(
        _mlp_body,
        grid=(BCH // BM,),
        in_specs=[
            pl.BlockSpec((NG, BM, 128), lambda i: (0, i, 0)),
            pl.BlockSpec((NG * 128, H1), lambda i: (0, 0)),
            pl.BlockSpec((1, H1), lambda i: (0, 0)),
            pl.BlockSpec((H1, H2), lambda i: (0, 0)),
            pl.BlockSpec((1, H2), lambda i: (0, 0)),
            pl.BlockSpec((H2, 1), lambda i: (0, 0)),
            pl.BlockSpec((1, 1), lambda i: (0, 0)),
        ],
        out_specs=pl.BlockSpec((BM, 1), lambda i: (i, 0)),
        out_shape=jax.ShapeDtypeStruct((BCH, 1), jnp.float32),
    )(xp, W1x, b1.reshape(1, H1), W2, b2.reshape(1, H2), W3,
      b3.reshape(1, 1))
    return out[:, 0]


def kernel(my_idx, ally_lists, enem_lists, misc_idx, emb_champ, emb_sp,
           emb_pri, emb_sub, emb_key, emb_pat, W1, b1, W2, b2, W3, b3):
    cidx = (
        jnp.concatenate([my_idx[None, :], ally_lists, enem_lists], axis=0)
        .astype(jnp.int32)
        .reshape(N_CSLOT * B)
    )
    mtab = jnp.concatenate([emb_sp, emb_pri, emb_sub, emb_key, emb_pat], axis=0)
    midx = (
        (misc_idx.astype(jnp.int32)
         + jnp.arange(N_MSLOT, dtype=jnp.int32)[None, :] * emb_sp.shape[0])
        .T.reshape(N_MSLOT * B)
    )
    xps = [
        _sc_gather(emb_champ, cidx, mtab, midx, c)
        for c in range(NCHUNK)
    ]
    outs = [_tc_mlp(xp, W1, b1, W2, b2, W3, b3) for xp in xps]
    return jnp.concatenate(outs, axis=0)


# speedup vs baseline: 1.2567x; 1.0018x over previous
import functools

import jax
import jax.numpy as jnp
from jax import lax
from jax.experimental import pallas as pl
from jax.experimental.pallas import tpu as pltpu, tpu_sc as plsc

B = 16384
D_CHAMP = 32
D_MISC = 16
H1 = 256
H2 = 128
N_CSLOT = 10
N_MSLOT = 5
NG = 4

NC = 2
NS = 16
NW = NC * NS
NCHUNK = 2
BCH = B // NCHUNK
BPW = BCH // NW
RING = 3

BM = 512


def _sc_gather(ctab, my_i, ally_i, enem_i, mtab, midx, chunk):
    mesh = plsc.VectorSubcoreMesh(core_axis_name="c", subcore_axis_name="s")

    @functools.partial(
        pl.kernel,
        mesh=mesh,
        compiler_params=pltpu.CompilerParams(use_tc_tiling_on_sc=False),
        out_type=jax.ShapeDtypeStruct((NG, BCH, 128), jnp.float32),
        scratch_types=[
            pltpu.VMEM((N_CSLOT, BPW), jnp.int32),
            pltpu.VMEM((N_MSLOT, BPW), jnp.int32),
            pltpu.VMEM((RING, BPW, D_CHAMP), jnp.float32),
            pltpu.VMEM((RING, BPW, D_MISC), jnp.float32),
            pltpu.SemaphoreType.DMA,
            pltpu.SemaphoreType.DMA,
            pltpu.SemaphoreType.DMA,
        ],
    )
    def k(ctab_hbm, my_hbm, ally_hbm, enem_hbm, mtab_hbm, midx_hbm, xp_hbm,
          idxc_v, idxm_v, crows_v, mrows_v, sg, sw, si):
        wid = lax.axis_index("s") * NC + lax.axis_index("c")
        base = wid * BPW
        cbase = chunk * BCH + base

        def idx_src(s):
            if s == 0:
                return my_hbm.at[pl.ds(cbase, BPW)]
            if s <= 4:
                return ally_hbm.at[s - 1, pl.ds(cbase, BPW)]
            return enem_hbm.at[s - 5, pl.ds(cbase, BPW)]

        idescs = []
        for s in range(N_CSLOT):
            idescs.append(pltpu.async_copy(idx_src(s), idxc_v.at[s], si))
        for m in range(N_MSLOT):
            idescs.append(pltpu.async_copy(
                midx_hbm.at[m, pl.ds(cbase, BPW)], idxm_v.at[m], si))

        def cgather(s):
            return pltpu.async_copy(
                ctab_hbm.at[idxc_v.at[s]],
                crows_v.at[s % RING], sg)

        def cwrite(s, lane, width=D_CHAMP, buf=None):
            return pltpu.async_copy(
                crows_v.at[buf if buf is not None else s % RING],
                xp_hbm.at[s // 4, pl.ds(base, BPW), pl.ds(lane, width)], sw)

        def mgather(m):
            return pltpu.async_copy(
                mtab_hbm.at[idxm_v.at[m]],
                mrows_v.at[m % RING], sg)

        def mwrite(m, lane, buf=None):
            return pltpu.async_copy(
                mrows_v.at[buf if buf is not None else m % RING],
                xp_hbm.at[3, pl.ds(base, BPW), pl.ds(lane, D_MISC)], sw)

        pend = []
        gd = {}
        for s in range(RING):
            idescs[s].wait()
            gd[s] = cgather(s)
        for s in range(RING, N_CSLOT):
            idescs[s].wait()
        for m in range(N_MSLOT):
            idescs[N_CSLOT + m].wait()
        for s in range(N_CSLOT):
            gd[s].wait()
            w = cwrite(s, (s % 4) * D_CHAMP)
            if s + RING < N_CSLOT:
                w.wait()
                gd[s + RING] = cgather(s + RING)
            else:
                pend.append(w)
        gm = {m: mgather(m) for m in range(RING)}
        pend.append(cwrite(8, 64, buf=8 % RING))
        pend.append(cwrite(8, 96, buf=8 % RING))
        for m in range(N_MSLOT):
            gm[m].wait()
            w = mwrite(m, m * D_MISC)
            if m + RING < N_MSLOT:
                w.wait()
                gm[m + RING] = mgather(m + RING)
            else:
                pend.append(w)
        pend.append(mwrite(4, 80, buf=4 % RING))
        pend.append(mwrite(4, 96, buf=4 % RING))
        pend.append(mwrite(4, 112, buf=4 % RING))
        for w in pend:
            w.wait()

    return k(ctab, my_i, ally_i, enem_i, mtab, midx)


def _mlp_body(xp_ref, w1x_ref, b1_ref, w2_ref, b2_ref, w3_ref, b3_ref,
              out_ref):
    x = jnp.concatenate(
        [xp_ref[0], xp_ref[1], xp_ref[2], xp_ref[3]], axis=-1)
    h = jnp.maximum(jnp.dot(x, w1x_ref[...]) + b1_ref[...], 0.0)
    h2 = jnp.maximum(jnp.dot(h, w2_ref[...]) + b2_ref[...], 0.0)
    out_ref[...] = jnp.dot(h2, w3_ref[...]) + b3_ref[...]


def _tc_mlp(xp, W1, b1, W2, b2, W3, b3):
    z64 = jnp.zeros((64, H1), jnp.float32)
    z48 = jnp.zeros((48, H1), jnp.float32)
    W1x = jnp.concatenate([W1[:320], z64, W1[320:], z48], axis=0)
    out = pl.pallas_call(
        _mlp_body,
        grid=(BCH // BM,),
        in_specs=[
            pl.BlockSpec((NG, BM, 128), lambda i: (0, i, 0)),
            pl.BlockSpec((NG * 128, H1), lambda i: (0, 0)),
            pl.BlockSpec((1, H1), lambda i: (0, 0)),
            pl.BlockSpec((H1, H2), lambda i: (0, 0)),
            pl.BlockSpec((1, H2), lambda i: (0, 0)),
            pl.BlockSpec((H2, 1), lambda i: (0, 0)),
            pl.BlockSpec((1, 1), lambda i: (0, 0)),
        ],
        out_specs=pl.BlockSpec((BM, 1), lambda i: (i, 0)),
        out_shape=jax.ShapeDtypeStruct((BCH, 1), jnp.float32),
    )(xp, W1x, b1.reshape(1, H1), W2, b2.reshape(1, H2), W3,
      b3.reshape(1, 1))
    return out[:, 0]


def kernel(my_idx, ally_lists, enem_lists, misc_idx, emb_champ, emb_sp,
           emb_pri, emb_sub, emb_key, emb_pat, W1, b1, W2, b2, W3, b3):
    mtab = jnp.concatenate([emb_sp, emb_pri, emb_sub, emb_key, emb_pat], axis=0)
    midx = (
        misc_idx.astype(jnp.int32)
        + jnp.arange(N_MSLOT, dtype=jnp.int32)[None, :] * emb_sp.shape[0]
    ).T
    xps = [
        _sc_gather(emb_champ, my_idx.astype(jnp.int32),
                   ally_lists.astype(jnp.int32), enem_lists.astype(jnp.int32),
                   mtab, midx, c)
        for c in range(NCHUNK)
    ]
    outs = [_tc_mlp(xp, W1, b1, W2, b2, W3, b3) for xp in xps]
    return jnp.concatenate(outs, axis=0)


# speedup vs baseline: 1.3864x; 1.1032x over previous
import functools

import jax
import jax.numpy as jnp
from jax import lax
from jax.experimental import pallas as pl
from jax.experimental.pallas import tpu as pltpu, tpu_sc as plsc

B = 16384
D_CHAMP = 32
D_MISC = 16
H1 = 256
H2 = 128
N_CSLOT = 10
N_MSLOT = 5
NG = 4

NC = 2
NS = 16
NW = NC * NS
NCHUNK = 2
BCH = B // NCHUNK
BPW = BCH // NW
RING = 3

BM = 512


def _sc_gather(ctab, my_i, ally_i, enem_i, mtab, midx, chunk):
    mesh = plsc.VectorSubcoreMesh(core_axis_name="c", subcore_axis_name="s")

    @functools.partial(
        pl.kernel,
        mesh=mesh,
        compiler_params=pltpu.CompilerParams(use_tc_tiling_on_sc=False),
        out_type=jax.ShapeDtypeStruct((NG, BCH, 128), jnp.float32),
        scratch_types=[
            pltpu.VMEM((N_CSLOT, BPW), jnp.int32),
            pltpu.VMEM((N_MSLOT, BPW), jnp.int32),
            pltpu.VMEM((RING, BPW, D_CHAMP), jnp.float32),
            pltpu.VMEM((RING, BPW, D_MISC), jnp.float32),
            pltpu.SemaphoreType.DMA,
            pltpu.SemaphoreType.DMA,
            pltpu.SemaphoreType.DMA,
        ],
    )
    def k(ctab_hbm, my_hbm, ally_hbm, enem_hbm, mtab_hbm, midx_hbm, xp_hbm,
          idxc_v, idxm_v, crows_v, mrows_v, sg, sw, si):
        wid = lax.axis_index("s") * NC + lax.axis_index("c")
        base = wid * BPW
        cbase = chunk * BCH + base

        def idx_src(s):
            if s == 0:
                return my_hbm.at[pl.ds(cbase, BPW)]
            if s <= 4:
                return ally_hbm.at[s - 1, pl.ds(cbase, BPW)]
            return enem_hbm.at[s - 5, pl.ds(cbase, BPW)]

        idescs = []
        for s in range(N_CSLOT):
            idescs.append(pltpu.async_copy(idx_src(s), idxc_v.at[s], si))
        for m in range(N_MSLOT):
            idescs.append(pltpu.async_copy(
                midx_hbm.at[m, pl.ds(cbase, BPW)], idxm_v.at[m], si))

        def cgather(s):
            return pltpu.async_copy(
                ctab_hbm.at[idxc_v.at[s]],
                crows_v.at[s % RING], sg)

        def cwrite(s, lane, width=D_CHAMP, buf=None):
            return pltpu.async_copy(
                crows_v.at[buf if buf is not None else s % RING],
                xp_hbm.at[s // 4, pl.ds(base, BPW), pl.ds(lane, width)], sw)

        def mgather(m):
            return pltpu.async_copy(
                mtab_hbm.at[idxm_v.at[m]],
                mrows_v.at[m % RING], sg)

        def mwrite(m, lane, buf=None):
            return pltpu.async_copy(
                mrows_v.at[buf if buf is not None else m % RING],
                xp_hbm.at[3, pl.ds(base, BPW), pl.ds(lane, D_MISC)], sw)

        pend = []
        gd = {}
        for s in range(RING):
            idescs[s].wait()
            gd[s] = cgather(s)
        for s in range(RING, N_CSLOT):
            idescs[s].wait()
        for m in range(N_MSLOT):
            idescs[N_CSLOT + m].wait()
        for s in range(N_CSLOT):
            gd[s].wait()
            w = cwrite(s, (s % 4) * D_CHAMP)
            if s + RING < N_CSLOT:
                w.wait()
                gd[s + RING] = cgather(s + RING)
            else:
                pend.append(w)
        gm = {m: mgather(m) for m in range(RING)}
        pend.append(cwrite(8, 64, buf=8 % RING))
        pend.append(cwrite(8, 96, buf=8 % RING))
        for m in range(N_MSLOT):
            gm[m].wait()
            w = mwrite(m, m * D_MISC)
            if m + RING < N_MSLOT:
                w.wait()
                gm[m + RING] = mgather(m + RING)
            else:
                pend.append(w)
        pend.append(mwrite(4, 80, buf=4 % RING))
        pend.append(mwrite(4, 96, buf=4 % RING))
        pend.append(mwrite(4, 112, buf=4 % RING))
        for w in pend:
            w.wait()

    return k(ctab, my_i, ally_i, enem_i, mtab, midx)


def _mlp_body(xp_ref, w1x_ref, b1_ref, w2_ref, b2_ref, w3_ref, b3_ref,
              out_ref):
    x = jnp.concatenate(
        [xp_ref[0], xp_ref[1], xp_ref[2], xp_ref[3]], axis=-1)
    h = jnp.maximum(jnp.dot(x, w1x_ref[...]) + b1_ref[...], 0.0)
    h2 = jnp.maximum(jnp.dot(h, w2_ref[...]) + b2_ref[...], 0.0)
    out_ref[...] = jnp.dot(h2, w3_ref[...]) + b3_ref[...]


def _tc_mlp(xp, W1, b1, W2, b2, W3, b3):
    z64 = jnp.zeros((64, H1), jnp.float32)
    z48 = jnp.zeros((48, H1), jnp.float32)
    W1x = jnp.concatenate([W1[:320], z64, W1[320:], z48], axis=0)
    out = pl.pallas_call(
        _mlp_body,
        grid=(BCH // BM,),
        in_specs=[
            pl.BlockSpec((NG, BM, 128), lambda i: (0, i, 0)),
            pl.BlockSpec((NG * 128, H1), lambda i: (0, 0)),
            pl.BlockSpec((1, H1), lambda i: (0, 0)),
            pl.BlockSpec((H1, H2), lambda i: (0, 0)),
            pl.BlockSpec((1, H2), lambda i: (0, 0)),
            pl.BlockSpec((H2, 1), lambda i: (0, 0)),
            pl.BlockSpec((1, 1), lambda i: (0, 0)),
        ],
        out_specs=pl.BlockSpec((BM, 1), lambda i: (i, 0)),
        out_shape=jax.ShapeDtypeStruct((BCH, 1), jnp.float32),
    )(xp, W1x, b1.reshape(1, H1), W2, b2.reshape(1, H2), W3,
      b3.reshape(1, 1))
    return out[:, 0]


def kernel(my_idx, ally_lists, enem_lists, misc_idx, emb_champ, emb_sp,
           emb_pri, emb_sub, emb_key, emb_pat, W1, b1, W2, b2, W3, b3):
    from jax.experimental.layout import Layout, with_layout_constraint
    emb_champ = with_layout_constraint(emb_champ, Layout((0, 1)))
    mtab = jnp.concatenate([emb_sp, emb_pri, emb_sub, emb_key, emb_pat], axis=0)
    midx = (
        misc_idx.astype(jnp.int32)
        + jnp.arange(N_MSLOT, dtype=jnp.int32)[None, :] * emb_sp.shape[0]
    ).T
    xps = [
        _sc_gather(emb_champ, my_idx.astype(jnp.int32),
                   ally_lists.astype(jnp.int32), enem_lists.astype(jnp.int32),
                   mtab, midx, c)
        for c in range(NCHUNK)
    ]
    outs = [_tc_mlp(xp, W1, b1, W2, b2, W3, b3) for xp in xps]
    return jnp.concatenate(outs, axis=0)
